# Initial kernel scaffold; baseline (speedup 1.0000x reference)
import jax
import jax.numpy as jnp
from jax.experimental import pallas as pl


def kernel(x, emb, W, b):
    raise NotImplementedError("write your pallas kernel here")


# speedup vs baseline: 12.6586x; 12.6586x over previous
import functools

import jax
import jax.numpy as jnp
from jax import lax
from jax.experimental import pallas as pl
from jax.experimental.pallas import tpu as pltpu
from jax.experimental.pallas import tpu_sc as plsc

_B = 16384
_S = 50
_V = 10000
_D = 128

_NC = 2
_NS = 16
_NW = _NC * _NS
_ROWS_PER_W = _B // _NW
_NB = 8
_CHUNKS = _ROWS_PER_W // _NB
_IDX_PER_CHUNK = _NB * _S
_LANES = 16


def _table_body(emb_ref, w_ref, b_ref, out_ref):
    j = pl.program_id(0)
    p = lax.dot_general(
        emb_ref[...], w_ref[...],
        dimension_numbers=(((1,), (1,)), ((), ())),
        preferred_element_type=jnp.float32,
    )
    out_ref[...] = p

    @pl.when(j == 0)
    def _():
        out_ref[...] = p + b_ref[...]


def _build_table(emb, W, b2d):
    return pl.pallas_call(
        _table_body,
        grid=(_S,),
        in_specs=[
            pl.BlockSpec((_V, _D), lambda j: (0, 0)),
            pl.BlockSpec((_D, _D), lambda j: (0, j)),
            pl.BlockSpec((1, _D), lambda j: (0, 0)),
        ],
        out_specs=pl.BlockSpec((_V, _D), lambda j: (j, 0)),
        out_shape=jax.ShapeDtypeStruct((_S * _V, _D), jnp.float32),
    )(emb, W, b2d)


def _gather_sum_body(p_hbm, x_hbm, out_hbm, idx_v, rows_v, out_v, sem0, sem1):
    wid = lax.axis_index("s") * _NC + lax.axis_index("c")
    row0 = wid * _ROWS_PER_W
    sems = (sem0, sem1)

    def load_idx(chunk, slot):
        pltpu.sync_copy(
            x_hbm.at[pl.ds(row0 + chunk * _NB, _NB)], idx_v.at[slot]
        )

    def start_gathers(slot):
        for r in range(_NB):
            pltpu.async_copy(
                p_hbm.at[idx_v.at[slot, r]],
                rows_v.at[slot, pl.ds(r * _S, _S)],
                sems[slot],
            )

    def wait_gathers(slot):
        pltpu.make_async_copy(
            p_hbm.at[pl.ds(0, _IDX_PER_CHUNK)], rows_v.at[slot], sems[slot]
        ).wait()

    def accumulate_and_store(chunk, slot):
        for r in range(_NB):
            base = r * _S

            def jstep(j, acc):
                return tuple(
                    acc[d] + rows_v[slot, base + j, pl.ds(d * _LANES, _LANES)]
                    for d in range(_D // _LANES)
                )

            acc = tuple(
                jnp.zeros((_LANES,), jnp.float32) for _ in range(_D // _LANES)
            )
            acc = lax.fori_loop(0, _S, jstep, acc, unroll=5)
            for d in range(_D // _LANES):
                out_v[r, pl.ds(d * _LANES, _LANES)] = acc[d]
        pltpu.sync_copy(
            out_v, out_hbm.at[pl.ds(row0 + chunk * _NB, _NB)]
        )

    load_idx(0, 0)
    start_gathers(0)

    @pl.loop(0, _CHUNKS, step=2)
    def _chunk_loop(c):
        for s in range(2):
            cc = c + s
            nslot = 1 - s

            @pl.when(cc + 1 < _CHUNKS)
            def _():
                load_idx(cc + 1, nslot)
                start_gathers(nslot)

            wait_gathers(s)
            accumulate_and_store(cc, s)


def _gather_sum(P, x2d):
    mesh = plsc.VectorSubcoreMesh(
        core_axis_name="c", subcore_axis_name="s",
        num_cores=_NC, num_subcores=_NS,
    )
    f = pl.kernel(
        _gather_sum_body,
        out_type=jax.ShapeDtypeStruct((_B, _D), jnp.float32),
        mesh=mesh,
        scratch_types=[
            pltpu.VMEM((2, _NB, _S), jnp.int32),
            pltpu.VMEM((2, _IDX_PER_CHUNK, _D), jnp.float32),
            pltpu.VMEM((_NB, _D), jnp.float32),
            pltpu.SemaphoreType.DMA,
            pltpu.SemaphoreType.DMA,
        ],
    )
    return f(P, x2d)


def kernel(x, emb, W, b):
    x = x.astype(jnp.int32)
    P = _build_table(emb, W, b.reshape(1, _D))
    xp = x + (_V * jnp.arange(_S, dtype=jnp.int32))[None, :]
    return _gather_sum(P, xp)
